# baseline (speedup 1.0000x reference)
import functools

import jax
import jax.numpy as jnp
from jax import lax
from jax.experimental import pallas as pl
from jax.experimental.pallas import tpu as pltpu
from jax.experimental.pallas import tpu_sc as plsc

N_NODES = 10000
N_EDGES = 320000
F_IN = 128
F_HID = 128
F_OUT = 64

NC = 2
NS = 16
NW = NC * NS
PER_W = N_EDGES // NW
CHUNK = 50
N_CHUNKS = PER_W // CHUNK
NBUF_BY_WIDTH = {128: 4, 64: 8}
NBUF_DEG = 8
SUB_ROWS = 624
ZROWS = 48
N_SLABS = SUB_ROWS // ZROWS
TAIL_OFF = NS * SUB_ROWS
TAIL = N_NODES - TAIL_OFF

_MESH = plsc.VectorSubcoreMesh(
    core_axis_name="c", subcore_axis_name="s", num_cores=NC, num_subcores=NS
)

_SC_PARAMS = pltpu.CompilerParams(use_tc_tiling_on_sc=False)


def _sc_segment_add(width, spmem_table=False):
  nbuf = NBUF_BY_WIDTH[width]
  n_bodies = N_CHUNKS // nbuf

  @functools.partial(
      pl.kernel,
      out_type=jax.ShapeDtypeStruct((NC, N_NODES, width), jnp.float32),
      mesh=_MESH,
      compiler_params=_SC_PARAMS,
      scratch_types=[
          pltpu.VMEM((N_CHUNKS, CHUNK), jnp.int32),
          pltpu.VMEM((N_CHUNKS, CHUNK), jnp.int32),
      ]
      + [pltpu.VMEM((CHUNK, width), jnp.float32) for _ in range(nbuf)]
      + ([pltpu.VMEM_SHARED((N_NODES, width), jnp.float32)]
         if spmem_table else [])
      + [
          pltpu.VMEM_SHARED((N_NODES, width), jnp.float32),
          pltpu.SemaphoreType.DMA,
          pltpu.SemaphoreType.DMA((nbuf,)),
          pltpu.SemaphoreType.DMA,
      ],
  )
  def k(h_hbm, src_hbm, dst_hbm, out_hbm, *rest):
    srcv, dstv = rest[0], rest[1]
    bufs = list(rest[2:2 + nbuf])
    rest = rest[2 + nbuf:]
    if spmem_table:
      tbl, acc, isem, gsem, ssem = rest
    else:
      acc, isem, gsem, ssem = rest
      tbl = None
    b0 = bufs[0]
    cid = lax.axis_index("c")
    sid = lax.axis_index("s")
    wid = sid * NC + cid

    di_s = pltpu.async_copy(src_hbm.at[wid], srcv, isem)
    di_d = pltpu.async_copy(dst_hbm.at[wid], dstv, isem)

    if spmem_table:
      pltpu.sync_copy(
          h_hbm.at[pl.ds(sid * SUB_ROWS, SUB_ROWS)],
          tbl.at[pl.ds(sid * SUB_ROWS, SUB_ROWS)],
      )
      @pl.when(sid == 0)
      def _():
        pltpu.sync_copy(
            h_hbm.at[pl.ds(TAIL_OFF, TAIL)], tbl.at[pl.ds(TAIL_OFF, TAIL)]
        )
    gather_src = tbl if spmem_table else h_hbm

    @pl.loop(0, ZROWS)
    def _(r):
      @pl.loop(0, width // 16)
      def _(c):
        b0[r, pl.ds(c * 16, 16)] = jnp.zeros((16,), jnp.float32)

    @pl.loop(0, N_SLABS)
    def _(i):
      pltpu.sync_copy(
          b0.at[pl.ds(0, ZROWS)],
          acc.at[pl.ds(sid * SUB_ROWS + i * ZROWS, ZROWS)],
      )

    @pl.when(sid == 0)
    def _():
      pltpu.sync_copy(b0.at[pl.ds(0, TAIL)], acc.at[pl.ds(TAIL_OFF, TAIL)])

    di_s.wait()
    di_d.wait()
    plsc.subcore_barrier()

    @pl.loop(0, n_bodies)
    def _(j):
      c0 = j * nbuf
      gds = [
          pltpu.async_copy(gather_src.at[srcv.at[c0 + b]], bufs[b], gsem.at[b])
          for b in range(nbuf)
      ]
      sds = []
      for b in range(nbuf):
        gds[b].wait()
        sds.append(
            pltpu.async_copy(bufs[b], acc.at[dstv.at[c0 + b]], ssem, add=True)
        )
      for d in sds:
        d.wait()

    plsc.subcore_barrier()

    pltpu.sync_copy(
        acc.at[pl.ds(sid * SUB_ROWS, SUB_ROWS)],
        out_hbm.at[cid].at[pl.ds(sid * SUB_ROWS, SUB_ROWS)],
    )

    @pl.when(sid == 0)
    def _():
      pltpu.sync_copy(
          acc.at[pl.ds(TAIL_OFF, TAIL)],
          out_hbm.at[cid].at[pl.ds(TAIL_OFF, TAIL)],
      )

  return k


_DEG_W = 16


@functools.partial(
    pl.kernel,
    out_type=jax.ShapeDtypeStruct((NC, N_NODES, _DEG_W), jnp.float32),
    mesh=_MESH,
    compiler_params=_SC_PARAMS,
    scratch_types=[
        pltpu.VMEM((N_CHUNKS, CHUNK), jnp.int32),
        pltpu.VMEM((CHUNK, _DEG_W), jnp.float32),
        pltpu.VMEM_SHARED((N_NODES, _DEG_W), jnp.float32),
        pltpu.SemaphoreType.DMA,
        pltpu.SemaphoreType.DMA,
    ],
)
def _sc_degree(dst_hbm, out_hbm, dstv, ones, acc, isem, ssem):
  cid = lax.axis_index("c")
  sid = lax.axis_index("s")
  wid = sid * NC + cid

  di = pltpu.async_copy(dst_hbm.at[wid], dstv, isem)

  @pl.loop(0, CHUNK)
  def _(r):
    ones[r, pl.ds(0, 16)] = jnp.zeros((16,), jnp.float32)

  @pl.loop(0, N_SLABS)
  def _(i):
    pltpu.sync_copy(
        ones.at[pl.ds(0, ZROWS)],
        acc.at[pl.ds(sid * SUB_ROWS + i * ZROWS, ZROWS)],
    )

  @pl.when(sid == 0)
  def _():
    pltpu.sync_copy(ones.at[pl.ds(0, TAIL)], acc.at[pl.ds(TAIL_OFF, TAIL)])

  @pl.loop(0, CHUNK)
  def _(r):
    ones[r, pl.ds(0, 16)] = jnp.ones((16,), jnp.float32)

  di.wait()
  plsc.subcore_barrier()

  @pl.loop(0, N_CHUNKS // NBUF_DEG)
  def _(j):
    sds = [
        pltpu.async_copy(ones, acc.at[dstv.at[j * NBUF_DEG + b]], ssem, add=True)
        for b in range(NBUF_DEG)
    ]
    for d in sds:
      d.wait()

  plsc.subcore_barrier()

  pltpu.sync_copy(
      acc.at[pl.ds(sid * SUB_ROWS, SUB_ROWS)],
      out_hbm.at[cid].at[pl.ds(sid * SUB_ROWS, SUB_ROWS)],
  )

  @pl.when(sid == 0)
  def _():
    pltpu.sync_copy(
        acc.at[pl.ds(TAIL_OFF, TAIL)],
        out_hbm.at[cid].at[pl.ds(TAIL_OFF, TAIL)],
    )


_BLK = 1000
_GRID = N_NODES // _BLK


def _tc_matmul(x, w):
  k, m = w.shape

  def body(x_ref, w_ref, o_ref):
    o_ref[...] = jnp.dot(
        x_ref[...], w_ref[...], preferred_element_type=jnp.float32
    )

  return pl.pallas_call(
      body,
      grid=(_GRID,),
      in_specs=[
          pl.BlockSpec((_BLK, k), lambda i: (i, 0)),
          pl.BlockSpec((k, m), lambda i: (0, 0)),
      ],
      out_specs=pl.BlockSpec((_BLK, m), lambda i: (i, 0)),
      out_shape=jax.ShapeDtypeStruct((N_NODES, m), jnp.float32),
  )(x, w)


def _dinv_of(degp_ref):
  d = degp_ref[0, :, 0] + degp_ref[1, :, 0] + 1.0
  return lax.rsqrt(d)[:, None]


def _tc_scale(degp, h):
  m = h.shape[1]

  def body(degp_ref, h_ref, o_ref):
    o_ref[...] = h_ref[...] * _dinv_of(degp_ref)

  return pl.pallas_call(
      body,
      grid=(_GRID,),
      in_specs=[
          pl.BlockSpec((NC, _BLK, _DEG_W), lambda i: (0, i, 0)),
          pl.BlockSpec((_BLK, m), lambda i: (i, 0)),
      ],
      out_specs=pl.BlockSpec((_BLK, m), lambda i: (i, 0)),
      out_shape=jax.ShapeDtypeStruct((N_NODES, m), jnp.float32),
  )(degp, h)


def _tc_layer1_finish(aggp, h1, degp, b1, w2):
  def body(aggp_ref, h1_ref, degp_ref, b1_ref, w2_ref, h2_ref, h2p_ref):
    dinv = _dinv_of(degp_ref)
    y = dinv * (aggp_ref[0] + aggp_ref[1]) + (dinv * dinv) * h1_ref[...]
    y = jnp.maximum(y + b1_ref[...][None, :], 0.0)
    h2 = jnp.dot(y, w2_ref[...], preferred_element_type=jnp.float32)
    h2_ref[...] = h2
    h2p_ref[...] = h2 * dinv

  return pl.pallas_call(
      body,
      grid=(_GRID,),
      in_specs=[
          pl.BlockSpec((NC, _BLK, F_HID), lambda i: (0, i, 0)),
          pl.BlockSpec((_BLK, F_HID), lambda i: (i, 0)),
          pl.BlockSpec((NC, _BLK, _DEG_W), lambda i: (0, i, 0)),
          pl.BlockSpec((F_HID,), lambda i: (0,)),
          pl.BlockSpec((F_HID, F_OUT), lambda i: (0, 0)),
      ],
      out_specs=[
          pl.BlockSpec((_BLK, F_OUT), lambda i: (i, 0)),
          pl.BlockSpec((_BLK, F_OUT), lambda i: (i, 0)),
      ],
      out_shape=[
          jax.ShapeDtypeStruct((N_NODES, F_OUT), jnp.float32),
          jax.ShapeDtypeStruct((N_NODES, F_OUT), jnp.float32),
      ],
  )(aggp, h1, degp, b1, w2)


def _tc_layer2_finish(aggp, h2, degp, b2):
  def body(aggp_ref, h2_ref, degp_ref, b2_ref, o_ref):
    dinv = _dinv_of(degp_ref)
    z = dinv * (aggp_ref[0] + aggp_ref[1]) + (dinv * dinv) * h2_ref[...]
    z = z + b2_ref[...][None, :]
    m = jnp.max(z, axis=1, keepdims=True)
    e = z - m
    o_ref[...] = e - jnp.log(jnp.sum(jnp.exp(e), axis=1, keepdims=True))

  return pl.pallas_call(
      body,
      grid=(_GRID,),
      in_specs=[
          pl.BlockSpec((NC, _BLK, F_OUT), lambda i: (0, i, 0)),
          pl.BlockSpec((_BLK, F_OUT), lambda i: (i, 0)),
          pl.BlockSpec((NC, _BLK, _DEG_W), lambda i: (0, i, 0)),
          pl.BlockSpec((F_OUT,), lambda i: (0,)),
      ],
      out_specs=pl.BlockSpec((_BLK, F_OUT), lambda i: (i, 0)),
      out_shape=jax.ShapeDtypeStruct((N_NODES, F_OUT), jnp.float32),
  )(aggp, h2, degp, b2)


_agg128 = _sc_segment_add(F_HID)
_agg64 = _sc_segment_add(F_OUT, spmem_table=True)


def kernel(x, edge_index, W1, b1, W2, b2):
  src = edge_index[0].astype(jnp.int32).reshape(NW, N_CHUNKS, CHUNK)
  dst = edge_index[1].astype(jnp.int32).reshape(NW, N_CHUNKS, CHUNK)

  degp = _sc_degree(dst)
  h1 = _tc_matmul(x, W1)
  h1p = _tc_scale(degp, h1)
  agg1 = _agg128(h1p, src, dst)
  h2, h2p = _tc_layer1_finish(agg1, h1, degp, b1, W2)
  agg2 = _agg64(h2p, src, dst)
  return _tc_layer2_finish(agg2, h2, degp, b2)


# speedup vs baseline: 1.0792x; 1.0792x over previous
import functools

import jax
import jax.numpy as jnp
from jax import lax
from jax.experimental import pallas as pl
from jax.experimental.pallas import tpu as pltpu
from jax.experimental.pallas import tpu_sc as plsc

N_NODES = 10000
N_EDGES = 320000
F_IN = 128
F_HID = 128
F_OUT = 64

NC = 2
NS = 16
NW = NC * NS
EBLK = 128
N_BLOCKS = N_EDGES // EBLK
NB = N_BLOCKS // NW
REM = N_BLOCKS - NB * NW
NBUF = 3
N_BODIES = NB // NBUF
SUB_ROWS = 624
ZROWS = 104
N_SLABS = SUB_ROWS // ZROWS
TAIL_OFF = NS * SUB_ROWS
TAIL = N_NODES - TAIL_OFF

_MESH = plsc.VectorSubcoreMesh(
    core_axis_name="c", subcore_axis_name="s", num_cores=NC, num_subcores=NS
)

_SC_PARAMS = pltpu.CompilerParams(use_tc_tiling_on_sc=False)


def _sc_segment_add(width, spmem_table=False):
  @functools.partial(
      pl.kernel,
      out_type=jax.ShapeDtypeStruct((NC, N_NODES, width), jnp.float32),
      mesh=_MESH,
      compiler_params=_SC_PARAMS,
      scratch_types=[pltpu.VMEM((2, EBLK), jnp.int32) for _ in range(2 * NBUF)]
      + [pltpu.VMEM((EBLK, width), jnp.float32) for _ in range(NBUF)]
      + ([pltpu.VMEM_SHARED((N_NODES, width), jnp.float32)]
         if spmem_table else [])
      + [
          pltpu.VMEM_SHARED((N_NODES, width), jnp.float32),
          pltpu.SemaphoreType.DMA,
          pltpu.SemaphoreType.DMA,
          pltpu.SemaphoreType.DMA((NBUF,)),
          pltpu.SemaphoreType.DMA,
      ],
  )
  def k(h_hbm, et_hbm, out_hbm, *rest):
    ia = list(rest[:NBUF])
    ib = list(rest[NBUF:2 * NBUF])
    bufs = list(rest[2 * NBUF:3 * NBUF])
    rest = rest[3 * NBUF:]
    if spmem_table:
      tbl, acc, isema, isemb, gsem, ssem = rest
    else:
      acc, isema, isemb, gsem, ssem = rest
      tbl = None
    b0 = bufs[0]
    cid = lax.axis_index("c")
    sid = lax.axis_index("s")
    wid = sid * NC + cid
    base = wid * NB

    for b in range(NBUF):
      pltpu.async_copy(et_hbm.at[base + b], ia[b], isema)

    if spmem_table:
      pltpu.sync_copy(
          h_hbm.at[pl.ds(sid * SUB_ROWS, SUB_ROWS)],
          tbl.at[pl.ds(sid * SUB_ROWS, SUB_ROWS)],
      )
      @pl.when(sid == 0)
      def _():
        pltpu.sync_copy(
            h_hbm.at[pl.ds(TAIL_OFF, TAIL)], tbl.at[pl.ds(TAIL_OFF, TAIL)]
        )
    gather_src = tbl if spmem_table else h_hbm

    @pl.loop(0, ZROWS)
    def _(r):
      @pl.loop(0, width // 16)
      def _(c):
        b0[r, pl.ds(c * 16, 16)] = jnp.zeros((16,), jnp.float32)

    @pl.loop(0, N_SLABS)
    def _(i):
      pltpu.sync_copy(
          b0.at[pl.ds(0, ZROWS)],
          acc.at[pl.ds(sid * SUB_ROWS + i * ZROWS, ZROWS)],
      )

    @pl.when(sid == 0)
    def _():
      pltpu.sync_copy(b0.at[pl.ds(0, TAIL)], acc.at[pl.ds(TAIL_OFF, TAIL)])

    plsc.subcore_barrier()

    def run_body(j, idx, prefetch):
      gds = [
          pltpu.async_copy(
              gather_src.at[idx[b].at[0]], bufs[b], gsem.at[b]
          )
          for b in range(NBUF)
      ]
      if prefetch is not None:
        nxt, other, sem = prefetch
        for b in range(NBUF):
          pltpu.async_copy(et_hbm.at[base + nxt * NBUF + b], other[b], sem)
      sds = []
      for b in range(NBUF):
        gds[b].wait()
        sds.append(
            pltpu.async_copy(bufs[b], acc.at[idx[b].at[1]], ssem, add=True)
        )
      for d in sds:
        d.wait()

    @pl.loop(0, N_BODIES // 2)
    def _(i):
      for b in range(NBUF):
        pltpu.make_async_copy(et_hbm.at[0], ia[b], isema).wait()
      run_body(2 * i, ia, (2 * i + 1, ib, isemb))
      for b in range(NBUF):
        pltpu.make_async_copy(et_hbm.at[0], ib[b], isemb).wait()

      @pl.when(i < N_BODIES // 2 - 1)
      def _():
        run_body(2 * i + 1, ib, (2 * i + 2, ia, isema))

      @pl.when(i == N_BODIES // 2 - 1)
      def _():
        run_body(2 * i + 1, ib, None)

    @pl.when(wid < REM)
    def _():
      pltpu.async_copy(et_hbm.at[NW * NB + wid], ia[0], isema).wait()
      pltpu.async_copy(gather_src.at[ia[0].at[0]], bufs[0], gsem.at[0]).wait()
      pltpu.async_copy(bufs[0], acc.at[ia[0].at[1]], ssem, add=True).wait()

    plsc.subcore_barrier()

    pltpu.sync_copy(
        acc.at[pl.ds(sid * SUB_ROWS, SUB_ROWS)],
        out_hbm.at[cid].at[pl.ds(sid * SUB_ROWS, SUB_ROWS)],
    )

    @pl.when(sid == 0)
    def _():
      pltpu.sync_copy(
          acc.at[pl.ds(TAIL_OFF, TAIL)],
          out_hbm.at[cid].at[pl.ds(TAIL_OFF, TAIL)],
      )

  return k


_DEG_W = 16


@functools.partial(
    pl.kernel,
    out_type=jax.ShapeDtypeStruct((NC, N_NODES, _DEG_W), jnp.float32),
    mesh=_MESH,
    compiler_params=_SC_PARAMS,
    scratch_types=[pltpu.VMEM((2, EBLK), jnp.int32) for _ in range(2 * NBUF)]
    + [
        pltpu.VMEM((EBLK, _DEG_W), jnp.float32),
        pltpu.VMEM_SHARED((N_NODES, _DEG_W), jnp.float32),
        pltpu.SemaphoreType.DMA,
        pltpu.SemaphoreType.DMA,
        pltpu.SemaphoreType.DMA,
    ],
)
def _sc_degree(et_hbm, out_hbm, *rest):
  ia = list(rest[:NBUF])
  ib = list(rest[NBUF:2 * NBUF])
  ones, acc, isema, isemb, ssem = rest[2 * NBUF:]
  cid = lax.axis_index("c")
  sid = lax.axis_index("s")
  wid = sid * NC + cid
  base = wid * NB

  for b in range(NBUF):
    pltpu.async_copy(et_hbm.at[base + b], ia[b], isema)

  @pl.loop(0, EBLK)
  def _(r):
    ones[r, pl.ds(0, 16)] = jnp.zeros((16,), jnp.float32)

  @pl.loop(0, N_SLABS)
  def _(i):
    pltpu.sync_copy(
        ones.at[pl.ds(0, ZROWS)],
        acc.at[pl.ds(sid * SUB_ROWS + i * ZROWS, ZROWS)],
    )

  @pl.when(sid == 0)
  def _():
    pltpu.sync_copy(ones.at[pl.ds(0, TAIL)], acc.at[pl.ds(TAIL_OFF, TAIL)])

  @pl.loop(0, EBLK)
  def _(r):
    ones[r, pl.ds(0, 16)] = jnp.ones((16,), jnp.float32)

  plsc.subcore_barrier()

  def deg_body(j, idx, prefetch):
    if prefetch is not None:
      nxt, other, sem = prefetch
      for b in range(NBUF):
        pltpu.async_copy(et_hbm.at[base + nxt * NBUF + b], other[b], sem)
    sds = [
        pltpu.async_copy(ones, acc.at[idx[b].at[1]], ssem, add=True)
        for b in range(NBUF)
    ]
    for d in sds:
      d.wait()

  @pl.loop(0, N_BODIES // 2)
  def _(i):
    for b in range(NBUF):
      pltpu.make_async_copy(et_hbm.at[0], ia[b], isema).wait()
    deg_body(2 * i, ia, (2 * i + 1, ib, isemb))
    for b in range(NBUF):
      pltpu.make_async_copy(et_hbm.at[0], ib[b], isemb).wait()

    @pl.when(i < N_BODIES // 2 - 1)
    def _():
      deg_body(2 * i + 1, ib, (2 * i + 2, ia, isema))

    @pl.when(i == N_BODIES // 2 - 1)
    def _():
      deg_body(2 * i + 1, ib, None)

  @pl.when(wid < REM)
  def _():
    pltpu.async_copy(et_hbm.at[NW * NB + wid], ia[0], isema).wait()
    pltpu.async_copy(ones, acc.at[ia[0].at[1]], ssem, add=True).wait()

  plsc.subcore_barrier()

  pltpu.sync_copy(
      acc.at[pl.ds(sid * SUB_ROWS, SUB_ROWS)],
      out_hbm.at[cid].at[pl.ds(sid * SUB_ROWS, SUB_ROWS)],
  )

  @pl.when(sid == 0)
  def _():
    pltpu.sync_copy(
        acc.at[pl.ds(TAIL_OFF, TAIL)],
        out_hbm.at[cid].at[pl.ds(TAIL_OFF, TAIL)],
    )


_BLK = 1000
_GRID = N_NODES // _BLK


def _tc_matmul(x, w):
  k, m = w.shape

  def body(x_ref, w_ref, o_ref):
    o_ref[...] = jnp.dot(
        x_ref[...], w_ref[...], preferred_element_type=jnp.float32
    )

  return pl.pallas_call(
      body,
      grid=(_GRID,),
      in_specs=[
          pl.BlockSpec((_BLK, k), lambda i: (i, 0)),
          pl.BlockSpec((k, m), lambda i: (0, 0)),
      ],
      out_specs=pl.BlockSpec((_BLK, m), lambda i: (i, 0)),
      out_shape=jax.ShapeDtypeStruct((N_NODES, m), jnp.float32),
  )(x, w)


def _tc_scale(degp, h):
  m = h.shape[1]

  def body(degp_ref, h_ref, o_ref, d128_ref, d64_ref):
    d = degp_ref[0, :, 0] + degp_ref[1, :, 0] + 1.0
    dinv = lax.rsqrt(d)[:, None]
    o_ref[...] = h_ref[...] * dinv
    d128_ref[...] = jnp.broadcast_to(dinv, (_BLK, F_HID))
    d64_ref[...] = jnp.broadcast_to(dinv, (_BLK, F_OUT))

  return pl.pallas_call(
      body,
      grid=(_GRID,),
      in_specs=[
          pl.BlockSpec((NC, _BLK, _DEG_W), lambda i: (0, i, 0)),
          pl.BlockSpec((_BLK, m), lambda i: (i, 0)),
      ],
      out_specs=[
          pl.BlockSpec((_BLK, m), lambda i: (i, 0)),
          pl.BlockSpec((_BLK, F_HID), lambda i: (i, 0)),
          pl.BlockSpec((_BLK, F_OUT), lambda i: (i, 0)),
      ],
      out_shape=[
          jax.ShapeDtypeStruct((N_NODES, m), jnp.float32),
          jax.ShapeDtypeStruct((N_NODES, F_HID), jnp.float32),
          jax.ShapeDtypeStruct((N_NODES, F_OUT), jnp.float32),
      ],
  )(degp, h)


def _tc_layer1_finish(aggp, h1, dvb, dvb64, b1, w2):
  def body(aggp_ref, h1_ref, dvb_ref, dvb64_ref, b1_ref, w2_ref,
           h2_ref, h2p_ref):
    dinv = dvb_ref[...]
    y = dinv * (aggp_ref[0] + aggp_ref[1]) + (dinv * dinv) * h1_ref[...]
    y = jnp.maximum(y + b1_ref[...][None, :], 0.0)
    h2 = jnp.dot(y, w2_ref[...], preferred_element_type=jnp.float32)
    h2_ref[...] = h2
    h2p_ref[...] = h2 * dvb64_ref[...]

  return pl.pallas_call(
      body,
      grid=(_GRID,),
      in_specs=[
          pl.BlockSpec((NC, _BLK, F_HID), lambda i: (0, i, 0)),
          pl.BlockSpec((_BLK, F_HID), lambda i: (i, 0)),
          pl.BlockSpec((_BLK, F_HID), lambda i: (i, 0)),
          pl.BlockSpec((_BLK, F_OUT), lambda i: (i, 0)),
          pl.BlockSpec((F_HID,), lambda i: (0,)),
          pl.BlockSpec((F_HID, F_OUT), lambda i: (0, 0)),
      ],
      out_specs=[
          pl.BlockSpec((_BLK, F_OUT), lambda i: (i, 0)),
          pl.BlockSpec((_BLK, F_OUT), lambda i: (i, 0)),
      ],
      out_shape=[
          jax.ShapeDtypeStruct((N_NODES, F_OUT), jnp.float32),
          jax.ShapeDtypeStruct((N_NODES, F_OUT), jnp.float32),
      ],
  )(aggp, h1, dvb, dvb64, b1, w2)


def _tc_layer2_finish(aggp, h2, dvb64, b2):
  def body(aggp_ref, h2_ref, dvb64_ref, b2_ref, o_ref):
    dinv = dvb64_ref[...]
    z = dinv * (aggp_ref[0] + aggp_ref[1]) + (dinv * dinv) * h2_ref[...]
    z = z + b2_ref[...][None, :]
    m = jnp.max(z, axis=1, keepdims=True)
    e = z - m
    o_ref[...] = e - jnp.log(jnp.sum(jnp.exp(e), axis=1, keepdims=True))

  return pl.pallas_call(
      body,
      grid=(_GRID,),
      in_specs=[
          pl.BlockSpec((NC, _BLK, F_OUT), lambda i: (0, i, 0)),
          pl.BlockSpec((_BLK, F_OUT), lambda i: (i, 0)),
          pl.BlockSpec((_BLK, F_OUT), lambda i: (i, 0)),
          pl.BlockSpec((F_OUT,), lambda i: (0,)),
      ],
      out_specs=pl.BlockSpec((_BLK, F_OUT), lambda i: (i, 0)),
      out_shape=jax.ShapeDtypeStruct((N_NODES, F_OUT), jnp.float32),
  )(aggp, h2, dvb64, b2)


_agg128 = _sc_segment_add(F_HID)
_agg64 = _sc_segment_add(F_OUT, spmem_table=True)


def kernel(x, edge_index, W1, b1, W2, b2):
  et = (
      edge_index.astype(jnp.int32)
      .reshape(2, N_BLOCKS, EBLK)
      .transpose(1, 0, 2)
  )

  degp = _sc_degree(et)
  h1 = _tc_matmul(x, W1)
  h1p, dvb, dvb64 = _tc_scale(degp, h1)
  agg1 = _agg128(h1p, et)
  h2, h2p = _tc_layer1_finish(agg1, h1, dvb, dvb64, b1, W2)
  agg2 = _agg64(h2p, et)
  return _tc_layer2_finish(agg2, h2, dvb64, b2)


# speedup vs baseline: 1.1530x; 1.0684x over previous
import functools

import jax
import jax.numpy as jnp
from jax import lax
from jax.experimental import pallas as pl
from jax.experimental.pallas import tpu as pltpu
from jax.experimental.pallas import tpu_sc as plsc

N_NODES = 10000
N_EDGES = 320000
F_IN = 128
F_HID = 128
F_OUT = 64

NC = 2
NS = 16
NW = NC * NS
EBLK = 128
N_BLOCKS = N_EDGES // EBLK
NB = N_BLOCKS // NW
REM = N_BLOCKS - NB * NW
NBUF = 3
N_BODIES = NB // NBUF
SUB_ROWS = 624
ZROWS = 104
N_SLABS = SUB_ROWS // ZROWS
TAIL_OFF = NS * SUB_ROWS
TAIL = N_NODES - TAIL_OFF

_MESH = plsc.VectorSubcoreMesh(
    core_axis_name="c", subcore_axis_name="s", num_cores=NC, num_subcores=NS
)

_SC_PARAMS = pltpu.CompilerParams(use_tc_tiling_on_sc=False)


def _sc_segment_add(width, spmem_table=False):
  packed = width == 64
  out_t = (
      jax.ShapeDtypeStruct((N_NODES, 2 * width), jnp.float32)
      if packed
      else jax.ShapeDtypeStruct((NC, N_NODES, width), jnp.float32)
  )

  @functools.partial(
      pl.kernel,
      out_type=out_t,
      mesh=_MESH,
      compiler_params=_SC_PARAMS,
      scratch_types=[pltpu.VMEM((2, EBLK), jnp.int32) for _ in range(2 * NBUF)]
      + [pltpu.VMEM((EBLK, width), jnp.float32) for _ in range(NBUF)]
      + ([pltpu.VMEM_SHARED((N_NODES, width), jnp.float32)]
         if spmem_table else [])
      + [
          pltpu.VMEM_SHARED((N_NODES, width), jnp.float32),
          pltpu.SemaphoreType.DMA,
          pltpu.SemaphoreType.DMA,
          pltpu.SemaphoreType.DMA((NBUF,)),
          pltpu.SemaphoreType.DMA,
      ],
  )
  def k(h_hbm, et_hbm, out_hbm, *rest):
    ia = list(rest[:NBUF])
    ib = list(rest[NBUF:2 * NBUF])
    bufs = list(rest[2 * NBUF:3 * NBUF])
    rest = rest[3 * NBUF:]
    if spmem_table:
      tbl, acc, isema, isemb, gsem, ssem = rest
    else:
      acc, isema, isemb, gsem, ssem = rest
      tbl = None
    b0 = bufs[0]
    cid = lax.axis_index("c")
    sid = lax.axis_index("s")
    wid = sid * NC + cid
    base = wid * NB

    for b in range(NBUF):
      pltpu.async_copy(et_hbm.at[base + b], ia[b], isema)

    if spmem_table:
      pltpu.sync_copy(
          h_hbm.at[pl.ds(sid * SUB_ROWS, SUB_ROWS)],
          tbl.at[pl.ds(sid * SUB_ROWS, SUB_ROWS)],
      )
      @pl.when(sid == 0)
      def _():
        pltpu.sync_copy(
            h_hbm.at[pl.ds(TAIL_OFF, TAIL)], tbl.at[pl.ds(TAIL_OFF, TAIL)]
        )
    gather_src = tbl if spmem_table else h_hbm

    @pl.loop(0, ZROWS)
    def _(r):
      @pl.loop(0, width // 16)
      def _(c):
        b0[r, pl.ds(c * 16, 16)] = jnp.zeros((16,), jnp.float32)

    @pl.loop(0, N_SLABS)
    def _(i):
      pltpu.sync_copy(
          b0.at[pl.ds(0, ZROWS)],
          acc.at[pl.ds(sid * SUB_ROWS + i * ZROWS, ZROWS)],
      )

    @pl.when(sid == 0)
    def _():
      pltpu.sync_copy(b0.at[pl.ds(0, TAIL)], acc.at[pl.ds(TAIL_OFF, TAIL)])

    plsc.subcore_barrier()

    def run_body(j, idx, prefetch):
      gds = [
          pltpu.async_copy(
              gather_src.at[idx[b].at[0]], bufs[b], gsem.at[b]
          )
          for b in range(NBUF)
      ]
      if prefetch is not None:
        nxt, other, sem = prefetch
        for b in range(NBUF):
          pltpu.async_copy(et_hbm.at[base + nxt * NBUF + b], other[b], sem)
      sds = []
      for b in range(NBUF):
        gds[b].wait()
        sds.append(
            pltpu.async_copy(bufs[b], acc.at[idx[b].at[1]], ssem, add=True)
        )
      for d in sds:
        d.wait()

    @pl.loop(0, N_BODIES // 2)
    def _(i):
      for b in range(NBUF):
        pltpu.make_async_copy(et_hbm.at[0], ia[b], isema).wait()
      run_body(2 * i, ia, (2 * i + 1, ib, isemb))
      for b in range(NBUF):
        pltpu.make_async_copy(et_hbm.at[0], ib[b], isemb).wait()

      @pl.when(i < N_BODIES // 2 - 1)
      def _():
        run_body(2 * i + 1, ib, (2 * i + 2, ia, isema))

      @pl.when(i == N_BODIES // 2 - 1)
      def _():
        run_body(2 * i + 1, ib, None)

    @pl.when(wid < REM)
    def _():
      pltpu.async_copy(et_hbm.at[NW * NB + wid], ia[0], isema).wait()
      pltpu.async_copy(gather_src.at[ia[0].at[0]], bufs[0], gsem.at[0]).wait()
      pltpu.async_copy(bufs[0], acc.at[ia[0].at[1]], ssem, add=True).wait()

    plsc.subcore_barrier()

    if packed:
      dst_full = out_hbm.at[pl.ds(sid * SUB_ROWS, SUB_ROWS),
                            pl.ds(cid * width, width)]
      dst_tail = out_hbm.at[pl.ds(TAIL_OFF, TAIL), pl.ds(cid * width, width)]
    else:
      dst_full = out_hbm.at[cid].at[pl.ds(sid * SUB_ROWS, SUB_ROWS)]
      dst_tail = out_hbm.at[cid].at[pl.ds(TAIL_OFF, TAIL)]
    pltpu.sync_copy(acc.at[pl.ds(sid * SUB_ROWS, SUB_ROWS)], dst_full)

    @pl.when(sid == 0)
    def _():
      pltpu.sync_copy(acc.at[pl.ds(TAIL_OFF, TAIL)], dst_tail)

  return k


_DEG_W = 16


@functools.partial(
    pl.kernel,
    out_type=jax.ShapeDtypeStruct((N_NODES, 128), jnp.float32),
    mesh=_MESH,
    compiler_params=_SC_PARAMS,
    scratch_types=[pltpu.VMEM((2, EBLK), jnp.int32) for _ in range(2 * NBUF)]
    + [
        pltpu.VMEM((EBLK, _DEG_W), jnp.float32),
        pltpu.VMEM_SHARED((N_NODES, _DEG_W), jnp.float32),
        pltpu.SemaphoreType.DMA,
        pltpu.SemaphoreType.DMA,
        pltpu.SemaphoreType.DMA,
    ],
)
def _sc_degree(et_hbm, out_hbm, *rest):
  ia = list(rest[:NBUF])
  ib = list(rest[NBUF:2 * NBUF])
  ones, acc, isema, isemb, ssem = rest[2 * NBUF:]
  cid = lax.axis_index("c")
  sid = lax.axis_index("s")
  wid = sid * NC + cid
  base = wid * NB

  for b in range(NBUF):
    pltpu.async_copy(et_hbm.at[base + b], ia[b], isema)

  @pl.loop(0, EBLK)
  def _(r):
    ones[r, pl.ds(0, 16)] = jnp.zeros((16,), jnp.float32)

  @pl.loop(0, N_SLABS)
  def _(i):
    pltpu.sync_copy(
        ones.at[pl.ds(0, ZROWS)],
        acc.at[pl.ds(sid * SUB_ROWS + i * ZROWS, ZROWS)],
    )

  @pl.when(sid == 0)
  def _():
    pltpu.sync_copy(ones.at[pl.ds(0, TAIL)], acc.at[pl.ds(TAIL_OFF, TAIL)])

  @pl.loop(0, EBLK)
  def _(r):
    ones[r, pl.ds(0, 16)] = jnp.ones((16,), jnp.float32)

  plsc.subcore_barrier()

  def deg_body(j, idx, prefetch):
    if prefetch is not None:
      nxt, other, sem = prefetch
      for b in range(NBUF):
        pltpu.async_copy(et_hbm.at[base + nxt * NBUF + b], other[b], sem)
    sds = [
        pltpu.async_copy(ones, acc.at[idx[b].at[1]], ssem, add=True)
        for b in range(NBUF)
    ]
    for d in sds:
      d.wait()

  @pl.loop(0, N_BODIES // 2)
  def _(i):
    for b in range(NBUF):
      pltpu.make_async_copy(et_hbm.at[0], ia[b], isema).wait()
    deg_body(2 * i, ia, (2 * i + 1, ib, isemb))
    for b in range(NBUF):
      pltpu.make_async_copy(et_hbm.at[0], ib[b], isemb).wait()

    @pl.when(i < N_BODIES // 2 - 1)
    def _():
      deg_body(2 * i + 1, ib, (2 * i + 2, ia, isema))

    @pl.when(i == N_BODIES // 2 - 1)
    def _():
      deg_body(2 * i + 1, ib, None)

  @pl.when(wid < REM)
  def _():
    pltpu.async_copy(et_hbm.at[NW * NB + wid], ia[0], isema).wait()
    pltpu.async_copy(ones, acc.at[ia[0].at[1]], ssem, add=True).wait()

  plsc.subcore_barrier()

  pltpu.sync_copy(
      acc.at[pl.ds(sid * SUB_ROWS, SUB_ROWS)],
      out_hbm.at[pl.ds(sid * SUB_ROWS, SUB_ROWS), pl.ds(cid * _DEG_W, _DEG_W)],
  )

  @pl.when(sid == 0)
  def _():
    pltpu.sync_copy(
        acc.at[pl.ds(TAIL_OFF, TAIL)],
        out_hbm.at[pl.ds(TAIL_OFF, TAIL), pl.ds(cid * _DEG_W, _DEG_W)],
    )


_BLK = 2000
_GRID = N_NODES // _BLK


def _tc_matmul(x, w):
  k, m = w.shape

  def body(x_ref, w_ref, o_ref):
    o_ref[...] = jnp.dot(
        x_ref[...], w_ref[...], preferred_element_type=jnp.float32
    )

  return pl.pallas_call(
      body,
      grid=(_GRID,),
      in_specs=[
          pl.BlockSpec((_BLK, k), lambda i: (i, 0)),
          pl.BlockSpec((k, m), lambda i: (0, 0)),
      ],
      out_specs=pl.BlockSpec((_BLK, m), lambda i: (i, 0)),
      out_shape=jax.ShapeDtypeStruct((N_NODES, m), jnp.float32),
  )(x, w)


def _tc_scale(degp, h):
  m = h.shape[1]

  def body(degp_ref, h_ref, o_ref, d128_ref, d64_ref):
    d = degp_ref[:, 0] + degp_ref[:, _DEG_W] + 1.0
    dinv = lax.rsqrt(d)[:, None]
    o_ref[...] = h_ref[...] * dinv
    d128_ref[...] = jnp.broadcast_to(dinv, (_BLK, F_HID))
    d64_ref[...] = jnp.broadcast_to(dinv, (_BLK, F_OUT))

  return pl.pallas_call(
      body,
      grid=(_GRID,),
      in_specs=[
          pl.BlockSpec((_BLK, 128), lambda i: (i, 0)),
          pl.BlockSpec((_BLK, m), lambda i: (i, 0)),
      ],
      out_specs=[
          pl.BlockSpec((_BLK, m), lambda i: (i, 0)),
          pl.BlockSpec((_BLK, F_HID), lambda i: (i, 0)),
          pl.BlockSpec((_BLK, F_OUT), lambda i: (i, 0)),
      ],
      out_shape=[
          jax.ShapeDtypeStruct((N_NODES, m), jnp.float32),
          jax.ShapeDtypeStruct((N_NODES, F_HID), jnp.float32),
          jax.ShapeDtypeStruct((N_NODES, F_OUT), jnp.float32),
      ],
  )(degp, h)


def _tc_layer1_finish(aggp, h1, dvb, dvb64, b1, w2):
  def body(p0_ref, p1_ref, h1_ref, dvb_ref, dvb64_ref, b1_ref, w2_ref,
           h2_ref, h2p_ref):
    dinv = dvb_ref[...]
    y = dinv * (p0_ref[...] + p1_ref[...]) + (dinv * dinv) * h1_ref[...]
    y = jnp.maximum(y + b1_ref[...][None, :], 0.0)
    h2 = jnp.dot(y, w2_ref[...], preferred_element_type=jnp.float32)
    h2_ref[...] = h2
    h2p_ref[...] = h2 * dvb64_ref[...]

  nblk = N_NODES // _BLK
  return pl.pallas_call(
      body,
      grid=(_GRID,),
      in_specs=[
          pl.BlockSpec((_BLK, F_HID), lambda i: (i, 0)),
          pl.BlockSpec((_BLK, F_HID), lambda i: (nblk + i, 0)),
          pl.BlockSpec((_BLK, F_HID), lambda i: (i, 0)),
          pl.BlockSpec((_BLK, F_HID), lambda i: (i, 0)),
          pl.BlockSpec((_BLK, F_OUT), lambda i: (i, 0)),
          pl.BlockSpec((F_HID,), lambda i: (0,)),
          pl.BlockSpec((F_HID, F_OUT), lambda i: (0, 0)),
      ],
      out_specs=[
          pl.BlockSpec((_BLK, F_OUT), lambda i: (i, 0)),
          pl.BlockSpec((_BLK, F_OUT), lambda i: (i, 0)),
      ],
      out_shape=[
          jax.ShapeDtypeStruct((N_NODES, F_OUT), jnp.float32),
          jax.ShapeDtypeStruct((N_NODES, F_OUT), jnp.float32),
      ],
  )(aggp, aggp, h1, dvb, dvb64, b1, w2)


def _tc_layer2_finish(aggp, h2, dvb64, b2):
  def body(aggp_ref, h2_ref, dvb64_ref, b2_ref, o_ref):
    dinv = dvb64_ref[...]
    p = aggp_ref[...]
    z = dinv * (p[:, :F_OUT] + p[:, F_OUT:]) + (dinv * dinv) * h2_ref[...]
    z = z + b2_ref[...][None, :]
    m = jnp.max(z, axis=1, keepdims=True)
    e = z - m
    o_ref[...] = e - jnp.log(jnp.sum(jnp.exp(e), axis=1, keepdims=True))

  return pl.pallas_call(
      body,
      grid=(_GRID,),
      in_specs=[
          pl.BlockSpec((_BLK, 128), lambda i: (i, 0)),
          pl.BlockSpec((_BLK, F_OUT), lambda i: (i, 0)),
          pl.BlockSpec((_BLK, F_OUT), lambda i: (i, 0)),
          pl.BlockSpec((F_OUT,), lambda i: (0,)),
      ],
      out_specs=pl.BlockSpec((_BLK, F_OUT), lambda i: (i, 0)),
      out_shape=jax.ShapeDtypeStruct((N_NODES, F_OUT), jnp.float32),
  )(aggp, h2, dvb64, b2)


_agg128 = _sc_segment_add(F_HID)
_agg64 = _sc_segment_add(F_OUT, spmem_table=True)


def kernel(x, edge_index, W1, b1, W2, b2):
  et = (
      edge_index.astype(jnp.int32)
      .reshape(2, N_BLOCKS, EBLK)
      .transpose(1, 0, 2)
  )

  degp = _sc_degree(et)
  h1 = _tc_matmul(x, W1)
  h1p, dvb, dvb64 = _tc_scale(degp, h1)
  agg1 = _agg128(h1p, et)
  agg1v = agg1.reshape(NC * N_NODES, F_HID)
  h2, h2p = _tc_layer1_finish(agg1v, h1, dvb, dvb64, b1, W2)
  agg2 = _agg64(h2p, et)
  return _tc_layer2_finish(agg2, h2, dvb64, b2)
